# baseline (speedup 1.0000x reference)
import functools

import jax
import jax.numpy as jnp
from jax.experimental import pallas as pl


def _slow_indices(T, alpha=4):
    n = T // alpha
    step = (T - 1) / (n - 1)
    return tuple(int(i * step) for i in range(n - 1)) + (T - 1,)


def _body(idx, T, in_ref, fast_ref, slow_ref):
    r = pl.program_id(0)
    t = jax.lax.rem(r, T)
    fast_ref[...] = in_ref[...]
    sel = functools.reduce(jnp.logical_or, [t == i for i in idx])

    @pl.when(sel)
    def _():
        slow_ref[...] = in_ref[...]


def kernel(frames):
    C, T, H, W = frames.shape
    idx = _slow_indices(T)
    S = len(idx)
    plane = H * W
    rows, lanes = plane // 128, 128

    x = frames.reshape(C * T, rows, lanes)

    def slow_j(t):
        return sum((t >= i).astype(jnp.int32) for i in idx[1:])

    def slow_map(r):
        t = jax.lax.rem(r, T)
        c = jax.lax.div(r, T)
        return (c * S + slow_j(t), 0, 0)

    fast, slow = pl.pallas_call(
        functools.partial(_body, idx, T),
        grid=(C * T,),
        in_specs=[pl.BlockSpec((1, rows, lanes), lambda r: (r, 0, 0))],
        out_specs=[
            pl.BlockSpec((1, rows, lanes), lambda r: (r, 0, 0)),
            pl.BlockSpec((1, rows, lanes), slow_map),
        ],
        out_shape=[
            jax.ShapeDtypeStruct((C * T, rows, lanes), frames.dtype),
            jax.ShapeDtypeStruct((C * S, rows, lanes), frames.dtype),
        ],
    )(x)

    return slow.reshape(C, S, H, W), fast.reshape(C, T, H, W)


# speedup vs baseline: 1.6525x; 1.6525x over previous
import functools

import jax
import jax.numpy as jnp
from jax import lax
from jax.experimental import pallas as pl
from jax.experimental.pallas import tpu as pltpu
from jax.experimental.pallas import tpu_sc as plsc

_C, _T, _H, _W = 3, 32, 224, 224
_S = _T // 4
_PLANE = _H * _W
_NC, _NS = 2, 16
_NW = _NC * _NS
_PARTS = 4
_CH = _PLANE // _PARTS
_CHUNKS = _C * _S * _PARTS
_PER_W = _CHUNKS // _NW


@functools.partial(
    pl.kernel,
    out_type=jax.ShapeDtypeStruct((_C * _S * _PLANE,), jnp.float32),
    mesh=plsc.VectorSubcoreMesh(core_axis_name="c", subcore_axis_name="s"),
    scratch_types=[
        [pltpu.VMEM((_CH,), jnp.float32) for _ in range(_PER_W)],
        pltpu.SemaphoreType.DMA,
    ],
)
def _sc_gather(frames_hbm, slow_hbm, bufs, sem):
    wid = lax.axis_index("s") * _NC + lax.axis_index("c")
    reads = []
    for k in range(_PER_W):
        g = wid * _PER_W + k
        plane = g // _PARTS
        part = g % _PARTS
        ch = plane // _S
        j = plane % _S
        t = (j * (_T - 1)) // (_S - 1)
        src = (ch * _T + t) * _PLANE + part * _CH
        reads.append(
            pltpu.async_copy(frames_hbm.at[pl.ds(src, _CH)], bufs[k], sem)
        )
    for d in reads:
        d.wait()
    writes = []
    for k in range(_PER_W):
        g = wid * _PER_W + k
        writes.append(
            pltpu.async_copy(bufs[k], slow_hbm.at[pl.ds(g * _CH, _CH)], sem)
        )
    for d in writes:
        d.wait()


def kernel(frames):
    slow = _sc_gather(frames.reshape(-1))
    return slow.reshape(_C, _S, _H, _W), frames
